# baseline (speedup 1.0000x reference)
import functools

import jax
import jax.numpy as jnp
from jax import lax
from jax.experimental import pallas as pl
from jax.experimental.pallas import tpu as pltpu
from jax.experimental.pallas import tpu_sc as plsc

ROWS = 128
N = 32768
L = 16
NV = N // L
NC = 2
NS = 16
NW = NC * NS
RPW = ROWS // NW

_PASSES = ((0, 11), (11, 11), (22, 10))
_HIST = 2048


def _desc_key(v):
    return jnp.where(v >= 0, v ^ 0x7FFFFFFF, v)


def _digit(k, sh, nb):
    d = lax.shift_right_logical(k, sh) if sh else k
    return jnp.bitwise_and(d, (1 << nb) - 1)


NCHUNK = 16
CV = NV // NCHUNK


def _sc_body(x_hbm, out_hbm, buf_a, buf_b, *hists):
    cid = lax.axis_index("c")
    sid = lax.axis_index("s")
    wid = sid * NC + cid

    zeros = jnp.zeros((L,), jnp.int32)

    for r in range(RPW):
        row = wid * RPW + r
        pltpu.sync_copy(x_hbm.at[row], buf_a)

        bufs = ((buf_a, buf_b), (buf_b, buf_a), (buf_a, buf_b))
        for p, (sh, nb) in enumerate(_PASSES):
            src, dst = bufs[p]
            first_pass = p == 0
            last_pass = p == len(_PASSES) - 1

            def zf(i, _):
                for h in hists:
                    h[pl.ds(i * L, L)] = zeros
                return 0
            lax.fori_loop(0, (1 << nb) // L, zf, 0, unroll=2)

            def hs(i, _):
                ks = []
                for c in range(NCHUNK):
                    off = (c * CV + i) * L
                    k = src[pl.ds(off, L)]
                    if first_pass:
                        k = _desc_key(k)
                        src[pl.ds(off, L)] = k
                    ks.append(k)
                scans = []
                for c in range(NCHUNK):
                    d = _digit(ks[c], sh, nb)
                    scans.append((d, *plsc.scan_count(d)))
                for c, h in enumerate(hists):
                    d, cnt, last = scans[c]
                    plsc.addupdate_scatter(h, [d], cnt, mask=last)
                return 0
            lax.fori_loop(0, CV, hs, 0, unroll=2)

            def pf(i, carry):
                vs = [h[pl.ds(i * L, L)] for h in hists]
                tot = vs[0]
                for v in vs[1:]:
                    tot = tot + v
                s = plsc.cumsum(tot)
                excl = s - tot + carry
                for c, h in enumerate(hists):
                    h[pl.ds(i * L, L)] = excl
                    if c + 1 < NCHUNK:
                        excl = excl + vs[c]
                return carry + jnp.sum(tot)
            lax.fori_loop(0, (1 << nb) // L, pf, jnp.int32(0))

            def pm(i, _):
                ks = [src[pl.ds((c * CV + i) * L, L)] for c in range(NCHUNK)]
                scans = []
                for c in range(NCHUNK):
                    d = _digit(ks[c], sh, nb)
                    scans.append((d, *plsc.scan_count(d)))
                bases = [plsc.load_gather(h, [scans[c][0]])
                         for c, h in enumerate(hists)]
                for c, h in enumerate(hists):
                    d, cnt, lastm = scans[c]
                    pos = bases[c] + cnt - 1
                    val = _desc_key(ks[c]) if last_pass else ks[c]
                    plsc.store_scatter(dst, [pos], val)
                    plsc.addupdate_scatter(h, [d], cnt, mask=lastm)
                return 0
            lax.fori_loop(0, CV, pm, 0, unroll=2)

        pltpu.sync_copy(buf_b, out_hbm.at[row])


@jax.jit
def kernel(x):
    xb = lax.bitcast_convert_type(x, jnp.int32)
    mesh = plsc.VectorSubcoreMesh(core_axis_name="c", subcore_axis_name="s")
    f = pl.kernel(
        _sc_body,
        out_type=jax.ShapeDtypeStruct((ROWS, N), jnp.int32),
        mesh=mesh,
        compiler_params=pltpu.CompilerParams(needs_layout_passes=False),
        scratch_types=[
            pltpu.VMEM((N,), jnp.int32),
            pltpu.VMEM((N,), jnp.int32),
        ] + [pltpu.VMEM((_HIST,), jnp.int32) for _ in range(NCHUNK)],
    )
    return lax.bitcast_convert_type(f(xb), jnp.float32)


# speedup vs baseline: 1.0401x; 1.0401x over previous
import functools

import jax
import jax.numpy as jnp
from jax import lax
from jax.experimental import pallas as pl
from jax.experimental.pallas import tpu as pltpu
from jax.experimental.pallas import tpu_sc as plsc

ROWS = 128
N = 32768
L = 16
NV = N // L
NC = 2
NS = 16
NW = NC * NS
RPW = ROWS // NW

_PASSES = ((0, 11), (11, 11), (22, 10))
_HIST = 2048


def _desc_key(v):
    return jnp.where(v >= 0, v ^ 0x7FFFFFFF, v)


def _digit(k, sh, nb):
    d = lax.shift_right_logical(k, sh) if sh else k
    return jnp.bitwise_and(d, (1 << nb) - 1)


NCHUNK = 16
CV = NV // NCHUNK


def _sc_body(x_hbm, out_hbm, buf_a, buf_b, *hists):
    cid = lax.axis_index("c")
    sid = lax.axis_index("s")
    wid = sid * NC + cid

    zeros = jnp.zeros((L,), jnp.int32)

    for r in range(RPW):
        row = wid * RPW + r
        pltpu.sync_copy(x_hbm.at[row], buf_a)

        bufs = ((buf_a, buf_b), (buf_b, buf_a), (buf_a, buf_b))
        for p, (sh, nb) in enumerate(_PASSES):
            src, dst = bufs[p]
            first_pass = p == 0
            last_pass = p == len(_PASSES) - 1

            def zf(i, _):
                for h in hists:
                    h[pl.ds(i * L, L)] = zeros
                return 0
            lax.fori_loop(0, (1 << nb) // L, zf, 0, unroll=4)

            def hs(i, _):
                ks = []
                for c in range(NCHUNK):
                    off = (c * CV + i) * L
                    k = src[pl.ds(off, L)]
                    if first_pass:
                        k = _desc_key(k)
                        src[pl.ds(off, L)] = k
                    ks.append(k)
                scans = []
                for c in range(NCHUNK):
                    d = _digit(ks[c], sh, nb)
                    scans.append((d, *plsc.scan_count(d)))
                for c, h in enumerate(hists):
                    d, cnt, last = scans[c]
                    plsc.addupdate_scatter(h, [d], cnt, mask=last)
                return 0
            lax.fori_loop(0, CV, hs, 0)

            def pf(i, carry):
                vs = [h[pl.ds(i * L, L)] for h in hists]
                tot = vs[0]
                for v in vs[1:]:
                    tot = tot + v
                s = plsc.cumsum(tot)
                excl = s - tot + carry
                for c, h in enumerate(hists):
                    h[pl.ds(i * L, L)] = excl
                    if c + 1 < NCHUNK:
                        excl = excl + vs[c]
                return carry + jnp.sum(tot)
            lax.fori_loop(0, (1 << nb) // L, pf, jnp.int32(0), unroll=4)

            def pm(i, _):
                ks = [src[pl.ds((c * CV + i) * L, L)] for c in range(NCHUNK)]
                scans = []
                for c in range(NCHUNK):
                    d = _digit(ks[c], sh, nb)
                    scans.append((d, *plsc.scan_count(d)))
                bases = [plsc.load_gather(h, [scans[c][0]])
                         for c, h in enumerate(hists)]
                for c, h in enumerate(hists):
                    d, cnt, lastm = scans[c]
                    pos = bases[c] + cnt - 1
                    val = _desc_key(ks[c]) if last_pass else ks[c]
                    plsc.store_scatter(dst, [pos], val)
                    plsc.addupdate_scatter(h, [d], cnt, mask=lastm)
                return 0
            lax.fori_loop(0, CV, pm, 0)

        pltpu.sync_copy(buf_b, out_hbm.at[row])


@jax.jit
def kernel(x):
    xb = lax.bitcast_convert_type(x, jnp.int32)
    mesh = plsc.VectorSubcoreMesh(core_axis_name="c", subcore_axis_name="s")
    f = pl.kernel(
        _sc_body,
        out_type=jax.ShapeDtypeStruct((ROWS, N), jnp.int32),
        mesh=mesh,
        compiler_params=pltpu.CompilerParams(needs_layout_passes=False),
        scratch_types=[
            pltpu.VMEM((N,), jnp.int32),
            pltpu.VMEM((N,), jnp.int32),
        ] + [pltpu.VMEM((_HIST,), jnp.int32) for _ in range(NCHUNK)],
    )
    return lax.bitcast_convert_type(f(xb), jnp.float32)


# speedup vs baseline: 1.0640x; 1.0230x over previous
import functools

import jax
import jax.numpy as jnp
from jax import lax
from jax.experimental import pallas as pl
from jax.experimental.pallas import tpu as pltpu
from jax.experimental.pallas import tpu_sc as plsc

ROWS = 128
N = 32768
L = 16
NV = N // L
NC = 2
NS = 16
NW = NC * NS
RPW = ROWS // NW

_PASSES = ((0, 11), (11, 11), (22, 10))
_HIST = 2048


def _desc_key(v):
    return jnp.where(v >= 0, v ^ 0x7FFFFFFF, v)


def _digit(k, sh, nb):
    d = lax.shift_right_logical(k, sh) if sh else k
    return jnp.bitwise_and(d, (1 << nb) - 1)


NCHUNK = 16
CV = NV // NCHUNK


def _sc_body(x_hbm, out_hbm, buf_a, buf_b, *hists):
    cid = lax.axis_index("c")
    sid = lax.axis_index("s")
    wid = sid * NC + cid

    zeros = jnp.zeros((L,), jnp.int32)

    for r in range(RPW):
        row = wid * RPW + r
        pltpu.sync_copy(x_hbm.at[row], buf_a)

        bufs = ((buf_a, buf_b), (buf_b, buf_a), (buf_a, buf_b))
        for p, (sh, nb) in enumerate(_PASSES):
            src, dst = bufs[p]
            first_pass = p == 0
            last_pass = p == len(_PASSES) - 1

            def zf(i, _):
                for h in hists:
                    h[pl.ds(i * L, L)] = zeros
                return 0
            lax.fori_loop(0, (1 << nb) // L, zf, 0, unroll=4)

            def hs(i, _):
                ks = []
                for c in range(NCHUNK):
                    off = (c * CV + i) * L
                    k = src[pl.ds(off, L)]
                    if first_pass:
                        k = _desc_key(k)
                        src[pl.ds(off, L)] = k
                    ks.append(k)
                ones = jnp.ones((L,), jnp.int32)
                ds = [_digit(ks[c], sh, nb) for c in range(NCHUNK)]
                for c, h in enumerate(hists):
                    plsc.addupdate_scatter(h, [ds[c]], ones)
                return 0
            lax.fori_loop(0, CV, hs, 0)

            def pf(i, carry):
                vs = [h[pl.ds(i * L, L)] for h in hists]
                tot = vs[0]
                for v in vs[1:]:
                    tot = tot + v
                s = plsc.cumsum(tot)
                excl = s - tot + carry
                for c, h in enumerate(hists):
                    h[pl.ds(i * L, L)] = excl
                    if c + 1 < NCHUNK:
                        excl = excl + vs[c]
                return carry + jnp.sum(tot)
            lax.fori_loop(0, (1 << nb) // L, pf, jnp.int32(0), unroll=4)

            def pm(i, _):
                ks = [src[pl.ds((c * CV + i) * L, L)] for c in range(NCHUNK)]
                scans = []
                for c in range(NCHUNK):
                    d = _digit(ks[c], sh, nb)
                    scans.append((d, *plsc.scan_count(d)))
                bases = [plsc.load_gather(h, [scans[c][0]])
                         for c, h in enumerate(hists)]
                for c, h in enumerate(hists):
                    d, cnt, lastm = scans[c]
                    pos = bases[c] + cnt - 1
                    val = _desc_key(ks[c]) if last_pass else ks[c]
                    plsc.store_scatter(dst, [pos], val)
                    plsc.addupdate_scatter(h, [d], cnt, mask=lastm)
                return 0
            lax.fori_loop(0, CV, pm, 0)

        pltpu.sync_copy(buf_b, out_hbm.at[row])


@jax.jit
def kernel(x):
    xb = lax.bitcast_convert_type(x, jnp.int32)
    mesh = plsc.VectorSubcoreMesh(core_axis_name="c", subcore_axis_name="s")
    f = pl.kernel(
        _sc_body,
        out_type=jax.ShapeDtypeStruct((ROWS, N), jnp.int32),
        mesh=mesh,
        compiler_params=pltpu.CompilerParams(needs_layout_passes=False),
        scratch_types=[
            pltpu.VMEM((N,), jnp.int32),
            pltpu.VMEM((N,), jnp.int32),
        ] + [pltpu.VMEM((_HIST,), jnp.int32) for _ in range(NCHUNK)],
    )
    return lax.bitcast_convert_type(f(xb), jnp.float32)
